# baseline (speedup 1.0000x reference)
import functools

import ml_dtypes
import numpy as np
import jax
import jax.numpy as jnp
from jax.experimental import pallas as pl
from jax.experimental.pallas import tpu as pltpu

_K_RATIO = 0.5
_KSIZE = 5
_SIGMA = 1.0


def _gauss_2d_bf16():
    c = _KSIZE // 2
    x, y = np.mgrid[0 - c:_KSIZE - c, 0 - c:_KSIZE - c]
    gk = 1.0 / (2.0 * np.pi * _SIGMA) * np.exp(
        -(np.square(x) + np.square(y)) / (2.0 * np.square(_SIGMA)))
    gk32 = gk.astype(np.float32)
    return gk32.astype(ml_dtypes.bfloat16).astype(np.float32)


def _batch_body(Bs, bands_ref, conf_ref, wm_ref, mask_ref, cnt_ref, pad_ref,
                cm_ref):
    L, A = conf_ref.shape[1], conf_ref.shape[2]
    H, W = mask_ref.shape[2], mask_ref.shape[3]
    NM = Bs * (L - 1)
    K = int(H * W * _K_RATIO)
    P = _KSIZE // 2
    b = pl.program_id(0)

    m = conf_ref[0, 1:, 0]
    for a in range(1, A):
        m = jnp.maximum(m, conf_ref[0, 1:, a])
    s = jax.nn.sigmoid(m) * wm_ref[0, 1:, 0]

    pad_ref[...] = jnp.zeros_like(pad_ref)
    pad_ref[:, P:P + H, P:P + W] = s.astype(jnp.bfloat16)
    for l in range(L - 1):
        acc = jnp.zeros((H, W), jnp.float32)
        for dy in range(_KSIZE):
            acc += jax.lax.dot_general(
                pad_ref[l, dy:dy + H, :], bands_ref[dy],
                (((1,), (0,)), ((), ())),
                preferred_element_type=jnp.float32)
        cm_ref[pl.ds(b * (L - 1) + l, 1)] = acc.reshape(1, H, W)

    @pl.when(b == Bs - 1)
    def _select():
        bits = jax.lax.bitcast_convert_type(cm_ref[...], jnp.int32)

        def step(_, lohi):
            lo, hi = lohi
            mid = lo + (hi - lo + 1) // 2
            cnt = jnp.sum((bits >= mid).astype(jnp.int32), axis=(1, 2),
                          keepdims=True)
            big = cnt >= K
            return jnp.where(big, mid, lo), jnp.where(big, hi, mid - 1)

        lo0 = jnp.zeros((NM, 1, 1), jnp.int32)
        hi0 = jnp.full((NM, 1, 1), 0x3F800000, jnp.int32)
        lo, _ = jax.lax.fori_loop(0, 30, step, (lo0, hi0))

        sel = (bits >= lo).astype(jnp.float32)
        cnt = jnp.sum(sel, axis=(1, 2)).reshape(Bs, L - 1, 1)
        cnt_ref[:, 0] = jnp.zeros((Bs, cnt_ref.shape[2]), jnp.float32)
        cnt_ref[:, 1:] = jnp.broadcast_to(cnt, (Bs, L - 1, cnt_ref.shape[2]))
        mask_ref[:, 0] = jnp.ones((Bs, H, W), jnp.float32)
        mask_ref[:, 1:] = sel.reshape(Bs, L - 1, H, W)


def kernel(batch_confidence_maps, B, batch_warp_maks_list, record_len,
           warp_vis_list, warp_conf_list, warp_x_list, gauss_kernel):
    Bs, L, A, H, W = batch_confidence_maps.shape
    P = _KSIZE // 2

    gw = _gauss_2d_bf16()
    bands_np = np.zeros((_KSIZE, W + 2 * P, W), np.float32)
    cols = np.arange(W)
    for dy in range(_KSIZE):
        for dx in range(_KSIZE):
            bands_np[dy, cols + dx, cols] = gw[dy, dx]
    bands = jnp.asarray(bands_np, dtype=jnp.bfloat16)

    masks, counts = pl.pallas_call(
        functools.partial(_batch_body, Bs),
        grid=(Bs,),
        in_specs=[
            pl.BlockSpec((_KSIZE, W + 2 * P, W), lambda b: (0, 0, 0)),
            pl.BlockSpec((1, L, A, H, W), lambda b: (b, 0, 0, 0, 0)),
            pl.BlockSpec((1, L, 1, H, W), lambda b: (b, 0, 0, 0, 0)),
        ],
        out_specs=[
            pl.BlockSpec((Bs, L, H, W), lambda b: (0, 0, 0, 0)),
            pl.BlockSpec((Bs, L, 128), lambda b: (0, 0, 0)),
        ],
        out_shape=[
            jax.ShapeDtypeStruct((Bs, L, H, W), jnp.float32),
            jax.ShapeDtypeStruct((Bs, L, 128), jnp.float32),
        ],
        scratch_shapes=[
            pltpu.VMEM((L - 1, H + 2 * P, W + 2 * P), jnp.bfloat16),
            pltpu.VMEM((Bs * (L - 1), H, W), jnp.float32),
        ],
        compiler_params=pltpu.CompilerParams(
            dimension_semantics=("arbitrary",)),
    )(bands, batch_confidence_maps, batch_warp_maks_list)

    masks = masks.reshape(Bs * L, 1, H, W)

    counts = counts[:, :, 0]
    rates = jnp.sum(counts[:, 1:], axis=1) / ((L - 1) * H * W)
    rate = jnp.sum(rates) / Bs
    return masks, rate


# speedup vs baseline: 1.0229x; 1.0229x over previous
import functools

import ml_dtypes
import numpy as np
import jax
import jax.numpy as jnp
from jax.experimental import pallas as pl
from jax.experimental.pallas import tpu as pltpu

_K_RATIO = 0.5
_KSIZE = 5
_SIGMA = 1.0


def _gauss_2d_bf16():
    c = _KSIZE // 2
    x, y = np.mgrid[0 - c:_KSIZE - c, 0 - c:_KSIZE - c]
    gk = 1.0 / (2.0 * np.pi * _SIGMA) * np.exp(
        -(np.square(x) + np.square(y)) / (2.0 * np.square(_SIGMA)))
    gk32 = gk.astype(np.float32)
    return gk32.astype(ml_dtypes.bfloat16).astype(np.float32)


def _batch_body(bands_ref, conf_ref, wm_ref, mask_ref, cnt_ref, pad_ref,
                cm_ref):
    G, L, A = conf_ref.shape[0], conf_ref.shape[1], conf_ref.shape[2]
    H, W = mask_ref.shape[2], mask_ref.shape[3]
    NM = G * (L - 1)
    K = int(H * W * _K_RATIO)
    P = _KSIZE // 2

    m = conf_ref[:, 1:, 0]
    for a in range(1, A):
        m = jnp.maximum(m, conf_ref[:, 1:, a])
    s = jax.nn.sigmoid(m) * wm_ref[:, 1:, 0]

    pad_ref[...] = jnp.zeros_like(pad_ref)
    pad_ref[:, P:P + H, P:P + W] = s.reshape(NM, H, W).astype(jnp.bfloat16)
    for l in range(NM):
        acc = jnp.zeros((H, W), jnp.float32)
        for dy in range(_KSIZE):
            acc += jax.lax.dot_general(
                pad_ref[l, dy:dy + H, :], bands_ref[dy],
                (((1,), (0,)), ((), ())),
                preferred_element_type=jnp.float32)
        cm_ref[l] = acc

    bits = jax.lax.bitcast_convert_type(cm_ref[...], jnp.int32)

    def step(_, lohi):
        lo, hi = lohi
        mid = lo + (hi - lo + 1) // 2
        cnt = jnp.sum((bits >= mid).astype(jnp.int32), axis=(1, 2),
                      keepdims=True)
        big = cnt >= K
        return jnp.where(big, mid, lo), jnp.where(big, hi, mid - 1)

    lo0 = jnp.zeros((NM, 1, 1), jnp.int32)
    hi0 = jnp.full((NM, 1, 1), 0x3F800000, jnp.int32)
    lo, _ = jax.lax.fori_loop(0, 30, step, (lo0, hi0))

    sel = (bits >= lo).astype(jnp.float32)
    cnt = jnp.sum(sel, axis=(1, 2)).reshape(G, L - 1, 1)
    cnt_ref[:, 0] = jnp.zeros((G, cnt_ref.shape[2]), jnp.float32)
    cnt_ref[:, 1:] = jnp.broadcast_to(cnt, (G, L - 1, cnt_ref.shape[2]))
    mask_ref[:, 0] = jnp.ones((G, H, W), jnp.float32)
    mask_ref[:, 1:] = sel.reshape(G, L - 1, H, W)


def kernel(batch_confidence_maps, B, batch_warp_maks_list, record_len,
           warp_vis_list, warp_conf_list, warp_x_list, gauss_kernel):
    Bs, L, A, H, W = batch_confidence_maps.shape
    P = _KSIZE // 2

    gw = _gauss_2d_bf16()
    bands_np = np.zeros((_KSIZE, W + 2 * P, W), np.float32)
    cols = np.arange(W)
    for dy in range(_KSIZE):
        for dx in range(_KSIZE):
            bands_np[dy, cols + dx, cols] = gw[dy, dx]
    bands = jnp.asarray(bands_np, dtype=jnp.bfloat16)

    G = Bs if Bs <= 4 else (2 if Bs % 2 == 0 else 1)
    masks, counts = pl.pallas_call(
        _batch_body,
        grid=(Bs // G,),
        in_specs=[
            pl.BlockSpec((_KSIZE, W + 2 * P, W), lambda b: (0, 0, 0)),
            pl.BlockSpec((G, L, A, H, W), lambda b: (b, 0, 0, 0, 0)),
            pl.BlockSpec((G, L, 1, H, W), lambda b: (b, 0, 0, 0, 0)),
        ],
        out_specs=[
            pl.BlockSpec((G, L, H, W), lambda b: (b, 0, 0, 0)),
            pl.BlockSpec((G, L, 128), lambda b: (b, 0, 0)),
        ],
        out_shape=[
            jax.ShapeDtypeStruct((Bs, L, H, W), jnp.float32),
            jax.ShapeDtypeStruct((Bs, L, 128), jnp.float32),
        ],
        scratch_shapes=[
            pltpu.VMEM((G * (L - 1), H + 2 * P, W + 2 * P), jnp.bfloat16),
            pltpu.VMEM((G * (L - 1), H, W), jnp.float32),
        ],
        compiler_params=pltpu.CompilerParams(
            dimension_semantics=("arbitrary",)),
    )(bands, batch_confidence_maps, batch_warp_maks_list)

    masks = masks.reshape(Bs * L, 1, H, W)

    counts = counts[:, :, 0]
    rates = jnp.sum(counts[:, 1:], axis=1) / ((L - 1) * H * W)
    rate = jnp.sum(rates) / Bs
    return masks, rate
